# baseline (speedup 1.0000x reference)
import functools

import jax
import jax.numpy as jnp
from jax import lax
from jax.experimental import pallas as pl
from jax.experimental.pallas import tpu as pltpu
from jax.experimental.pallas import tpu_sc as plsc

VOCAB = 1000000
MAX_LEN = 200
EMBED_DIM = 64
BATCH = 4096

B_FLAT = BATCH * MAX_LEN
NUM_CORES = 2
NUM_SUBCORES = 16
NW = NUM_CORES * NUM_SUBCORES
B_PER_W = B_FLAT // NW
CHUNK = 1600
N_CHUNKS = B_PER_W // CHUNK
IDX_MINOR = 100
N_STREAMS = CHUNK // IDX_MINOR
SEQ_PER_CHUNK = CHUNK // MAX_LEN
VECS = EMBED_DIM // 16


def _build():
  mesh = plsc.VectorSubcoreMesh(core_axis_name="c", subcore_axis_name="s")

  @functools.partial(
      pl.kernel,
      mesh=mesh,
      compiler_params=pltpu.CompilerParams(use_tc_tiling_on_sc=False),
      out_type=jax.ShapeDtypeStruct((B_FLAT, EMBED_DIM), jnp.float32),
      scratch_types=[
          pltpu.VMEM((N_STREAMS, IDX_MINOR), jnp.int32),
          pltpu.VMEM((CHUNK, EMBED_DIM), jnp.float32),
          pltpu.VMEM((MAX_LEN, EMBED_DIM), jnp.float32),
          pltpu.SemaphoreType.DMA,
      ],
  )
  def emb_kernel(idx_hbm, table_hbm, pos_hbm, out_hbm, idx_v, rows_v, pos_v,
                 sem):
    wid = lax.axis_index("s") * NUM_CORES + lax.axis_index("c")
    row_base = wid * B_PER_W
    idxrow_base = wid * (B_PER_W // IDX_MINOR)

    pltpu.sync_copy(pos_hbm, pos_v)

    def chunk_body(k, carry):
      row0 = row_base + k * CHUNK
      idxrow0 = idxrow_base + k * N_STREAMS

      pltpu.sync_copy(idx_hbm.at[pl.ds(idxrow0, N_STREAMS)], idx_v)

      copies = []
      for j in range(N_STREAMS):
        copies.append(
            pltpu.async_copy(
                table_hbm.at[idx_v.at[j]],
                rows_v.at[pl.ds(j * IDX_MINOR, IDX_MINOR)],
                sem,
            ))
      for c in copies:
        c.wait()

      def add_body(l, c2):
        for t in range(VECS):
          p = pos_v[l, pl.ds(16 * t, 16)]
          for s in range(SEQ_PER_CHUNK):
            r = s * MAX_LEN + l
            rows_v[r, pl.ds(16 * t, 16)] += p
        return c2

      lax.fori_loop(0, MAX_LEN, add_body, 0)

      pltpu.sync_copy(rows_v, out_hbm.at[pl.ds(row0, CHUNK)])
      return carry

    lax.fori_loop(0, N_CHUNKS, chunk_body, 0)

  return emb_kernel


_emb = _build()


def kernel(inputs, token_table, pos_table):
  idx = inputs.reshape(-1).astype(jnp.int32)
  idx2d = idx.reshape(B_FLAT // IDX_MINOR, IDX_MINOR)
  out = _emb(idx2d, token_table, pos_table)
  return out.reshape(BATCH, MAX_LEN, EMBED_DIM)


# speedup vs baseline: 1.0359x; 1.0359x over previous
import functools

import jax
import jax.numpy as jnp
from jax import lax
from jax.experimental import pallas as pl
from jax.experimental.pallas import tpu as pltpu
from jax.experimental.pallas import tpu_sc as plsc

VOCAB = 1000000
MAX_LEN = 200
EMBED_DIM = 64
BATCH = 4096

B_FLAT = BATCH * MAX_LEN
NUM_CORES = 2
NUM_SUBCORES = 16
NW = NUM_CORES * NUM_SUBCORES
B_PER_W = B_FLAT // NW
CHUNK = 800
N_CHUNKS = B_PER_W // CHUNK
IDX_MINOR = 100
N_STREAMS = CHUNK // IDX_MINOR
VECS = EMBED_DIM // 16


def _build():
  mesh = plsc.VectorSubcoreMesh(core_axis_name="c", subcore_axis_name="s")

  @functools.partial(
      pl.kernel,
      mesh=mesh,
      compiler_params=pltpu.CompilerParams(use_tc_tiling_on_sc=False),
      out_type=jax.ShapeDtypeStruct((B_FLAT, EMBED_DIM), jnp.float32),
      scratch_types=[
          pltpu.VMEM((N_STREAMS, IDX_MINOR), jnp.int32),
          pltpu.VMEM((N_STREAMS, IDX_MINOR), jnp.int32),
          pltpu.VMEM((CHUNK, EMBED_DIM), jnp.float32),
          pltpu.VMEM((CHUNK, EMBED_DIM), jnp.float32),
          pltpu.VMEM((MAX_LEN, EMBED_DIM), jnp.float32),
          pltpu.SemaphoreType.DMA,
          pltpu.SemaphoreType.DMA,
          pltpu.SemaphoreType.DMA,
          pltpu.SemaphoreType.DMA,
      ],
  )
  def emb_kernel(idx_hbm, table_hbm, pos_hbm, out_hbm,
                 idx_a, idx_b, rows_a, rows_b, pos_v,
                 gsem_a, gsem_b, wsem_a, wsem_b):
    wid = lax.axis_index("s") * NUM_CORES + lax.axis_index("c")
    row_base = wid * B_PER_W
    idxrow_base = wid * (B_PER_W // IDX_MINOR)

    pltpu.sync_copy(pos_hbm, pos_v)

    bufs = ((idx_a, rows_a, gsem_a, wsem_a),
            (idx_b, rows_b, gsem_b, wsem_b))

    def fire(buf, k):
      idx_v, rows_v, gsem, _ = buf
      pltpu.sync_copy(idx_hbm.at[pl.ds(idxrow_base + k * N_STREAMS, N_STREAMS)],
                      idx_v)
      for j in range(N_STREAMS):
        pltpu.async_copy(
            table_hbm.at[idx_v.at[j]],
            rows_v.at[pl.ds(j * IDX_MINOR, IDX_MINOR)],
            gsem,
        )

    def wait_write(buf):
      _, rows_v, _, wsem = buf
      pltpu.make_async_copy(rows_v, out_hbm.at[pl.ds(0, CHUNK)], wsem).wait()

    def process(buf):
      idx_v, rows_v, gsem, _ = buf
      for j in range(N_STREAMS):
        pltpu.make_async_copy(
            table_hbm.at[idx_v.at[j]],
            rows_v.at[pl.ds(j * IDX_MINOR, IDX_MINOR)],
            gsem,
        ).wait()
        l_base = (j * IDX_MINOR) % MAX_LEN

        def add_body(l, c, j=j, l_base=l_base):
          r = j * IDX_MINOR + l
          for t in range(VECS):
            p = pos_v[l_base + l, pl.ds(16 * t, 16)]
            plsc.addupdate(rows_v.at[r, pl.ds(16 * t, 16)], p)
          return c

        lax.fori_loop(0, IDX_MINOR, add_body, 0)

    def write(buf, k):
      _, rows_v, _, wsem = buf
      pltpu.async_copy(rows_v, out_hbm.at[pl.ds(row_base + k * CHUNK, CHUNK)],
                       wsem)

    fire(bufs[0], 0)
    fire(bufs[1], 1)

    def pair_body(i, carry):
      k = 2 * i
      process(bufs[0])
      write(bufs[0], k)
      process(bufs[1])
      write(bufs[1], k + 1)
      wait_write(bufs[0])
      fire(bufs[0], k + 2)
      wait_write(bufs[1])
      fire(bufs[1], k + 3)
      return carry

    lax.fori_loop(0, N_CHUNKS // 2 - 1, pair_body, 0)

    process(bufs[0])
    write(bufs[0], N_CHUNKS - 2)
    process(bufs[1])
    write(bufs[1], N_CHUNKS - 1)
    wait_write(bufs[0])
    wait_write(bufs[1])

  return emb_kernel


_emb = _build()


def kernel(inputs, token_table, pos_table):
  idx = inputs.reshape(-1).astype(jnp.int32)
  idx2d = idx.reshape(B_FLAT // IDX_MINOR, IDX_MINOR)
  out = _emb(idx2d, token_table, pos_table)
  return out.reshape(BATCH, MAX_LEN, EMBED_DIM)
